# Initial kernel scaffold; baseline (speedup 1.0000x reference)
import jax
import jax.numpy as jnp
from jax.experimental import pallas as pl


def kernel(router_logits):
    raise NotImplementedError("write your pallas kernel here")


# speedup vs baseline: 4.0809x; 4.0809x over previous
import jax
import jax.numpy as jnp
from jax import lax
from jax.experimental import pallas as pl
from jax.experimental.pallas import tpu as pltpu
from jax.experimental.pallas import tpu_sc as plsc

_T = 32768
_E = 64
_L = 16
_NC = 2
_NS = 16
_NW = _NC * _NS
_TPW = _T // _NW
_CH = 256
_NCH = _TPW // _CH
_U = 16


def _router_sc_body(logits_hbm, combine_hbm, idx_hbm, part_hbm,
                    in_v, out_v, idx_v, part_v):
    wid = lax.axis_index("s") * _NC + lax.axis_index("c")
    base = wid * _TPW

    lane = lax.iota(jnp.int32, _L)
    ec = [lane + jnp.int32(16 * j) for j in range(4)]
    big = jnp.full((_L,), jnp.int32(_E), jnp.int32)
    onev = jnp.full((_L,), jnp.float32(1.0), jnp.float32)
    zerov = jnp.zeros((_L,), jnp.float32)

    def token(t, u, acc):
        p0, p1, p2, p3, c0, c1, c2, c3, ivec = acc
        v = [in_v[t, pl.ds(16 * j, _L)] for j in range(4)]
        m = jnp.max(jnp.maximum(jnp.maximum(v[0], v[1]),
                                jnp.maximum(v[2], v[3])))
        cand = [jnp.where(v[j] == m, ec[j], big) for j in range(4)]
        idx = jnp.min(jnp.minimum(jnp.minimum(cand[0], cand[1]),
                                  jnp.minimum(cand[2], cand[3])))
        ex = [jnp.exp(v[j] - m) for j in range(4)]
        s = jnp.sum(ex[0] + ex[1] + ex[2] + ex[3])
        inv = onev / (zerov + s)
        p = [ex[j] * inv for j in range(4)]
        msk = [ec[j] == idx for j in range(4)]
        for j in range(4):
            out_v[t, pl.ds(16 * j, _L)] = jnp.where(msk[j], p[j], zerov)
        ivec = jnp.where(lane == jnp.int32(u), idx, ivec)
        return (p0 + p[0], p1 + p[1], p2 + p[2], p3 + p[3],
                c0 + jnp.where(msk[0], onev, zerov),
                c1 + jnp.where(msk[1], onev, zerov),
                c2 + jnp.where(msk[2], onev, zerov),
                c3 + jnp.where(msk[3], onev, zerov),
                ivec)

    def chunk_body(i, acc):
        acc = acc + (big,)
        for u in range(_U):
            acc = token(i * _U + u, u, acc)
        idx_v[pl.ds(i * _U, _U)] = acc[8]
        return acc[:8]

    acc = (zerov,) * 8
    for c in range(_NCH):
        row0 = base + c * _CH
        pltpu.sync_copy(logits_hbm.at[pl.ds(row0, _CH)], in_v)
        acc = lax.fori_loop(0, _CH // _U, chunk_body, acc)
        pltpu.sync_copy(out_v, combine_hbm.at[pl.ds(row0, _CH)])
        pltpu.sync_copy(idx_v, idx_hbm.at[pl.ds(row0, _CH)])
    for j in range(4):
        part_v[0, pl.ds(16 * j, _L)] = acc[4 + j]
        part_v[0, pl.ds(_E + 16 * j, _L)] = acc[j]
    pltpu.sync_copy(part_v, part_hbm.at[pl.ds(wid, 1)])


def _aux_tc_body(part_ref, aux_ref):
    x = part_ref[...]
    cs = jnp.sum(x[:, :_E], axis=0)
    ps = jnp.sum(x[:, _E:], axis=0)
    scale = jnp.float32(_E) / (jnp.float32(_T) * jnp.float32(_T))
    aux_ref[0, 0] = scale * jnp.sum(cs * ps)


def kernel(router_logits):
    combine, idx, part = pl.kernel(
        _router_sc_body,
        out_type=[
            jax.ShapeDtypeStruct((_T, _E), jnp.float32),
            jax.ShapeDtypeStruct((_T,), jnp.int32),
            jax.ShapeDtypeStruct((_NW, 2 * _E), jnp.float32),
        ],
        mesh=plsc.VectorSubcoreMesh(core_axis_name="c", subcore_axis_name="s",
                                    num_cores=_NC, num_subcores=_NS),
        compiler_params=pltpu.CompilerParams(needs_layout_passes=False),
        scratch_types=[
            pltpu.VMEM((_CH, _E), jnp.float32),
            pltpu.VMEM((_CH, _E), jnp.float32),
            pltpu.VMEM((_CH,), jnp.int32),
            pltpu.VMEM((1, 2 * _E), jnp.float32),
        ],
    )(router_logits)
    aux = pl.pallas_call(
        _aux_tc_body,
        out_shape=jax.ShapeDtypeStruct((1, 1), jnp.float32),
        out_specs=pl.BlockSpec(memory_space=pltpu.SMEM),
    )(part)[0, 0]
    return combine, idx, aux


# speedup vs baseline: 4.0894x; 1.0021x over previous
import jax
import jax.numpy as jnp
from jax import lax
from jax.experimental import pallas as pl
from jax.experimental.pallas import tpu as pltpu
from jax.experimental.pallas import tpu_sc as plsc

_T = 32768
_E = 64
_L = 16
_NC = 2
_NS = 16
_NW = _NC * _NS
_TPW = _T // _NW
_CH = 256
_NCH = _TPW // _CH
_U = 16


def _router_sc_body(logits_hbm, combine_hbm, idx_hbm, part_hbm,
                    in_v, out_v, idx_v, part_v):
    wid = lax.axis_index("s") * _NC + lax.axis_index("c")
    base = wid * _TPW

    lane = lax.iota(jnp.int32, _L)
    ec = [lane + jnp.int32(16 * j) for j in range(4)]
    big = jnp.full((_L,), jnp.int32(_E), jnp.int32)
    onev = jnp.full((_L,), jnp.float32(1.0), jnp.float32)
    zerov = jnp.zeros((_L,), jnp.float32)

    def token(t, u, acc):
        p0, p1, p2, p3, c0, c1, c2, c3, ivec = acc
        v = [in_v[t, pl.ds(16 * j, _L)] for j in range(4)]
        m = jnp.max(jnp.maximum(jnp.maximum(v[0], v[1]),
                                jnp.maximum(v[2], v[3])))
        cand = [jnp.where(v[j] == m, ec[j], big) for j in range(4)]
        idx = jnp.min(jnp.minimum(jnp.minimum(cand[0], cand[1]),
                                  jnp.minimum(cand[2], cand[3])))
        ex = [jnp.exp(v[j] - m) for j in range(4)]
        s = jnp.sum(ex[0] + ex[1] + ex[2] + ex[3])
        inv = onev / (zerov + s)
        p = [ex[j] * inv for j in range(4)]
        msk = [ec[j] == idx for j in range(4)]
        for j in range(4):
            out_v[t, pl.ds(16 * j, _L)] = jnp.where(msk[j], p[j], zerov)
        ivec = jnp.where(lane == jnp.int32(u), idx, ivec)
        return (p0 + p[0], p1 + p[1], p2 + p[2], p3 + p[3],
                c0 + jnp.where(msk[0], onev, zerov),
                c1 + jnp.where(msk[1], onev, zerov),
                c2 + jnp.where(msk[2], onev, zerov),
                c3 + jnp.where(msk[3], onev, zerov),
                ivec)

    def chunk_body(i, acc):
        acc = acc + (big,)
        for u in range(_U):
            acc = token(i * _U + u, u, acc)
        idx_v[pl.ds(i * _U, _U)] = acc[8]
        return acc[:8]

    acc = (zerov,) * 8
    for c in range(_NCH):
        row0 = base + c * _CH
        pltpu.sync_copy(logits_hbm.at[pl.ds(row0, _CH)], in_v)
        acc = lax.fori_loop(0, _CH // _U, chunk_body, acc)
        pltpu.sync_copy(out_v, combine_hbm.at[pl.ds(row0, _CH)])
        pltpu.sync_copy(idx_v, idx_hbm.at[pl.ds(row0, _CH)])
    for j in range(4):
        part_v[0, pl.ds(16 * j, _L)] = acc[4 + j]
        part_v[0, pl.ds(_E + 16 * j, _L)] = acc[j]
    pltpu.sync_copy(part_v, part_hbm.at[pl.ds(wid, 1)])


def _aux_tc_body(part_ref, aux_ref):
    x = part_ref[...]
    cs = jnp.sum(x[:, :_E], axis=0)
    ps = jnp.sum(x[:, _E:], axis=0)
    scale = jnp.float32(_E) / (jnp.float32(_T) * jnp.float32(_T))
    aux_ref[0, 0] = scale * jnp.sum(cs * ps)


def kernel(router_logits):
    combine, idx, part = pl.kernel(
        _router_sc_body,
        out_type=[
            jax.ShapeDtypeStruct((_T, _E), jnp.float32),
            jax.ShapeDtypeStruct((_T,), jnp.int32),
            jax.ShapeDtypeStruct((_NW, 2 * _E), jnp.float32),
        ],
        mesh=plsc.VectorSubcoreMesh(core_axis_name="c", subcore_axis_name="s",
                                    num_cores=_NC, num_subcores=_NS),
        compiler_params=pltpu.CompilerParams(needs_layout_passes=False,
                                             use_tc_tiling_on_sc=True),
        scratch_types=[
            pltpu.VMEM((_CH, _E), jnp.float32),
            pltpu.VMEM((_CH, _E), jnp.float32),
            pltpu.VMEM((_CH,), jnp.int32),
            pltpu.VMEM((1, 2 * _E), jnp.float32),
        ],
    )(router_logits)
    aux = pl.pallas_call(
        _aux_tc_body,
        out_shape=jax.ShapeDtypeStruct((1, 1), jnp.float32),
        out_specs=pl.BlockSpec(memory_space=pltpu.SMEM),
    )(part)[0, 0]
    return combine, idx, aux
